# Initial kernel scaffold; baseline (speedup 1.0000x reference)
import jax
import jax.numpy as jnp
from jax.experimental import pallas as pl


def kernel(positions, local_or_ghost, W1, b1, W2, b2, edge_index, batch, ptr):
    raise NotImplementedError("write your pallas kernel here")


# speedup vs baseline: 9.2175x; 9.2175x over previous
import functools

import jax
import jax.numpy as jnp
from jax import lax
from jax.experimental import pallas as pl
from jax.experimental.pallas import tpu as pltpu
from jax.experimental.pallas import tpu_sc as plsc

N_NODES = 100000
N_EDGES = 1600000
NUM_GRAPHS = 64
N_RBF = 8
HIDDEN = 16

NPAD = 100352
CHUNK = 512
CHUNKS_PER_TILE = 98
EPT = CHUNK * CHUNKS_PER_TILE
EPAD = EPT * 32
CENTERS = [0.0 + 5.0 * k / (N_RBF - 1) for k in range(N_RBF)]

_mesh = plsc.VectorSubcoreMesh(core_axis_name="c", subcore_axis_name="s")
_params = pltpu.CompilerParams(needs_layout_passes=False,
                               use_tc_tiling_on_sc=False)


def _bcast_i32(x):
    return jnp.broadcast_to(jnp.int32(x), (16,))


def _rsqrt(u):
    bits = plsc.bitcast(u, jnp.int32)
    y = plsc.bitcast(jnp.int32(0x5F3759DF) - (bits >> 1), jnp.float32)
    for _ in range(3):
        y = y * (1.5 - 0.5 * u * y * y)
    return y


@functools.partial(
    pl.kernel,
    out_type=jax.ShapeDtypeStruct((8 * NPAD,), jnp.float32),
    mesh=_mesh,
    compiler_params=_params,
    scratch_types=[
        pltpu.VMEM((176,), jnp.float32),
        pltpu.VMEM((4, 128), jnp.int32),
        pltpu.VMEM((4, 128), jnp.int32),
        pltpu.VMEM((CHUNK,), jnp.float32),
        pltpu.VMEM((CHUNK,), jnp.float32),
        pltpu.VMEM((CHUNK,), jnp.float32),
        pltpu.VMEM((CHUNK,), jnp.float32),
        pltpu.VMEM((CHUNK,), jnp.float32),
        pltpu.VMEM((CHUNK,), jnp.float32),
        pltpu.VMEM((CHUNK,), jnp.float32),
        pltpu.VMEM((CHUNK,), jnp.float32),
        pltpu.VMEM((CHUNK,), jnp.float32),
        pltpu.VMEM((CHUNK,), jnp.float32),
        pltpu.VMEM((CHUNK,), jnp.float32),
        pltpu.VMEM((CHUNK,), jnp.float32),
        pltpu.VMEM((CHUNK,), jnp.float32),
        pltpu.VMEM_SHARED((NPAD,), jnp.float32),
        pltpu.VMEM_SHARED((NPAD,), jnp.float32),
        pltpu.VMEM_SHARED((NPAD,), jnp.float32),
        pltpu.VMEM_SHARED((NPAD,), jnp.float32),
        pltpu.SemaphoreType.DMA,
    ],
)
def _edge_kernel(px_h, py_h, pz_h, srcix_h, dstix_h, wpack_h, zeros_h, out_h,
                 wp, sidx, didx, sxb, syb, szb, dxb, dyb, dzb,
                 eb, gxb, gyb, gzb, nxb, nyb, nzb,
                 ae, agx, agy, agz, sem):
    cid = lax.axis_index("c")
    sid = lax.axis_index("s")
    wid = cid * 16 + sid

    zr = NPAD // 16
    zsl = pl.ds(sid * zr, zr)
    pltpu.sync_copy(zeros_h.at[zsl], ae.at[zsl])
    pltpu.sync_copy(zeros_h.at[zsl], agx.at[zsl])
    pltpu.sync_copy(zeros_h.at[zsl], agy.at[zsl])
    pltpu.sync_copy(zeros_h.at[zsl], agz.at[zsl])
    pltpu.sync_copy(wpack_h, wp)
    plsc.subcore_barrier()

    wvecs = [wp[pl.ds(i * 16, 16)] for i in range(11)]
    s_w1 = [[wvecs[k][jj] for jj in range(16)] for k in range(N_RBF)]
    s_b1 = [wvecs[8][jj] for jj in range(16)]
    s_w2 = [wvecs[9][jj] for jj in range(16)]
    s_b2 = wvecs[10][0]

    def chunk_body(c, carry):
        base = wid * EPT + c * CHUNK
        for j in range(4):
            pltpu.sync_copy(srcix_h.at[pl.ds(base + j * 128, 128)], sidx.at[j])
            pltpu.sync_copy(dstix_h.at[pl.ds(base + j * 128, 128)], didx.at[j])
        cps = []
        for j in range(4):
            bsl = pl.ds(j * 128, 128)
            cps.append(pltpu.async_copy(px_h.at[sidx.at[j]], sxb.at[bsl], sem))
            cps.append(pltpu.async_copy(py_h.at[sidx.at[j]], syb.at[bsl], sem))
            cps.append(pltpu.async_copy(pz_h.at[sidx.at[j]], szb.at[bsl], sem))
            cps.append(pltpu.async_copy(px_h.at[didx.at[j]], dxb.at[bsl], sem))
            cps.append(pltpu.async_copy(py_h.at[didx.at[j]], dyb.at[bsl], sem))
            cps.append(pltpu.async_copy(pz_h.at[didx.at[j]], dzb.at[bsl], sem))
        for cp in cps:
            cp.wait()

        def grp(g, carry2):
            gsl = pl.ds(g * 16, 16)
            vx = dxb[gsl] - sxb[gsl]
            vy = dyb[gsl] - syb[gsl]
            vz = dzb[gsl] - szb[gsl]
            u = vx * vx + vy * vy + vz * vz + 1e-9
            inv_d = _rsqrt(u)
            d = u * inv_d

            rbf = []
            drbf = []
            for k in range(N_RBF):
                t = d - CENTERS[k]
                r = jnp.exp(-(t * t))
                rbf.append(r)
                drbf.append(-2.0 * t * r)

            e_vec = None
            dd_vec = None
            for jj in range(HIDDEN):
                zv = rbf[0] * s_w1[0][jj]
                gv = drbf[0] * s_w1[0][jj]
                for k in range(1, N_RBF):
                    zv = zv + rbf[k] * s_w1[k][jj]
                    gv = gv + drbf[k] * s_w1[k][jj]
                zv = zv + s_b1[jj]
                ex = jnp.exp(zv + zv)
                th = (ex - 1.0) / (ex + 1.0)
                w2s = s_w2[jj]
                et = th * w2s
                dt = (1.0 - th * th) * gv * w2s
                e_vec = et if e_vec is None else e_vec + et
                dd_vec = dt if dd_vec is None else dd_vec + dt
            e_vec = e_vec + s_b2

            coef = dd_vec * inv_d
            gx = coef * vx
            gy = coef * vy
            gz = coef * vz

            eb[gsl] = e_vec
            gxb[gsl] = gx
            gyb[gsl] = gy
            gzb[gsl] = gz
            nxb[gsl] = -gx
            nyb[gsl] = -gy
            nzb[gsl] = -gz
            return carry2

        lax.fori_loop(0, CHUNK // 16, grp, 0)

        for j in range(4):
            bsl = pl.ds(j * 128, 128)
            pltpu.sync_copy(eb.at[bsl], ae.at[didx.at[j]], add=True)
            pltpu.sync_copy(gxb.at[bsl], agx.at[didx.at[j]], add=True)
            pltpu.sync_copy(gyb.at[bsl], agy.at[didx.at[j]], add=True)
            pltpu.sync_copy(gzb.at[bsl], agz.at[didx.at[j]], add=True)
            pltpu.sync_copy(nxb.at[bsl], agx.at[sidx.at[j]], add=True)
            pltpu.sync_copy(nyb.at[bsl], agy.at[sidx.at[j]], add=True)
            pltpu.sync_copy(nzb.at[bsl], agz.at[sidx.at[j]], add=True)
        return carry

    lax.fori_loop(0, CHUNKS_PER_TILE, chunk_body, 0)

    plsc.subcore_barrier()
    obase = cid * 4 * NPAD
    pltpu.sync_copy(ae.at[zsl], out_h.at[pl.ds(obase + sid * zr, zr)])
    pltpu.sync_copy(agx.at[zsl], out_h.at[pl.ds(obase + NPAD + sid * zr, zr)])
    pltpu.sync_copy(agy.at[zsl], out_h.at[pl.ds(obase + 2 * NPAD + sid * zr, zr)])
    pltpu.sync_copy(agz.at[zsl], out_h.at[pl.ds(obase + 3 * NPAD + sid * zr, zr)])


_ROWS_B = NPAD // 32


@functools.partial(
    pl.kernel,
    out_type=(
        jax.ShapeDtypeStruct((4 * NPAD,), jnp.float32),
        jax.ShapeDtypeStruct((2 * NUM_GRAPHS,), jnp.float32),
    ),
    mesh=_mesh,
    compiler_params=_params,
    scratch_types=[
        pltpu.VMEM((_ROWS_B,), jnp.float32),
        pltpu.VMEM((_ROWS_B,), jnp.float32),
        pltpu.VMEM((_ROWS_B,), jnp.float32),
        pltpu.VMEM((_ROWS_B,), jnp.float32),
        pltpu.VMEM((_ROWS_B,), jnp.float32),
        pltpu.VMEM((_ROWS_B,), jnp.float32),
        pltpu.VMEM((_ROWS_B,), jnp.float32),
        pltpu.VMEM((_ROWS_B,), jnp.float32),
        pltpu.VMEM((_ROWS_B,), jnp.float32),
        pltpu.VMEM((_ROWS_B,), jnp.int32),
        pltpu.VMEM((_ROWS_B,), jnp.float32),
        pltpu.VMEM((_ROWS_B,), jnp.float32),
        pltpu.VMEM((_ROWS_B,), jnp.float32),
        pltpu.VMEM((_ROWS_B,), jnp.float32),
        pltpu.VMEM((NUM_GRAPHS * 16,), jnp.float32),
        pltpu.VMEM((16 * NUM_GRAPHS * 16,), jnp.float32),
        pltpu.VMEM((NUM_GRAPHS,), jnp.float32),
        pltpu.VMEM_SHARED((16 * NUM_GRAPHS * 16,), jnp.float32),
    ],
)
def _node_kernel(p_h, log_h, batch_h, out_h, tot_h,
                 pae, pax, pay, paz, pbe, pbx, pby, pbz, lg, bt,
                 oe, ofx, ofy, ofz, bkt, tbuf, t64, sbkt):
    cid = lax.axis_index("c")
    sid = lax.axis_index("s")
    wid = cid * 16 + sid
    nb = wid * _ROWS_B
    sl = pl.ds(nb, _ROWS_B)

    pltpu.sync_copy(p_h.at[pl.ds(0 * NPAD + nb, _ROWS_B)], pae)
    pltpu.sync_copy(p_h.at[pl.ds(1 * NPAD + nb, _ROWS_B)], pax)
    pltpu.sync_copy(p_h.at[pl.ds(2 * NPAD + nb, _ROWS_B)], pay)
    pltpu.sync_copy(p_h.at[pl.ds(3 * NPAD + nb, _ROWS_B)], paz)
    pltpu.sync_copy(p_h.at[pl.ds(4 * NPAD + nb, _ROWS_B)], pbe)
    pltpu.sync_copy(p_h.at[pl.ds(5 * NPAD + nb, _ROWS_B)], pbx)
    pltpu.sync_copy(p_h.at[pl.ds(6 * NPAD + nb, _ROWS_B)], pby)
    pltpu.sync_copy(p_h.at[pl.ds(7 * NPAD + nb, _ROWS_B)], pbz)
    pltpu.sync_copy(log_h.at[sl], lg)
    pltpu.sync_copy(batch_h.at[sl], bt)

    lane = lax.iota(jnp.int32, 16)
    zero_v = jnp.broadcast_to(jnp.float32(0.0), (16,))

    def zb(i, carry):
        bkt[pl.ds(i * 16, 16)] = zero_v
        return carry

    lax.fori_loop(0, NUM_GRAPHS, zb, 0)

    def grp(g, carry):
        gsl = pl.ds(g * 16, 16)
        e = pae[gsl] + pbe[gsl]
        fx = -(pax[gsl] + pbx[gsl])
        fy = -(pay[gsl] + pby[gsl])
        fz = -(paz[gsl] + pbz[gsl])
        nel = e * lg[gsl]
        bv = bt[gsl]
        plsc.addupdate_scatter(bkt, [bv * 16 + lane], nel)
        oe[gsl] = e
        ofx[gsl] = fx
        ofy[gsl] = fy
        ofz[gsl] = fz
        return carry

    lax.fori_loop(0, _ROWS_B // 16, grp, 0)

    pltpu.sync_copy(oe, out_h.at[pl.ds(0 * NPAD + nb, _ROWS_B)])
    pltpu.sync_copy(ofx, out_h.at[pl.ds(1 * NPAD + nb, _ROWS_B)])
    pltpu.sync_copy(ofy, out_h.at[pl.ds(2 * NPAD + nb, _ROWS_B)])
    pltpu.sync_copy(ofz, out_h.at[pl.ds(3 * NPAD + nb, _ROWS_B)])
    pltpu.sync_copy(bkt, sbkt.at[pl.ds(sid * NUM_GRAPHS * 16, NUM_GRAPHS * 16)])
    plsc.subcore_barrier()

    @pl.when(sid == 0)
    def _():
        pltpu.sync_copy(sbkt, tbuf)
        for jg in range(NUM_GRAPHS):
            s = tbuf[pl.ds(jg * 16, 16)]
            for t in range(1, 16):
                s = s + tbuf[pl.ds(t * NUM_GRAPHS * 16 + jg * 16, 16)]
            plsc.store_scatter(t64, [_bcast_i32(jg)],
                               jnp.broadcast_to(jnp.sum(s), (16,)))
        pltpu.sync_copy(t64, tot_h.at[pl.ds(cid * NUM_GRAPHS, NUM_GRAPHS)])


def kernel(positions, local_or_ghost, W1, b1, W2, b2, edge_index, batch, ptr):
    f32 = jnp.float32
    npd = NPAD - N_NODES
    px = jnp.pad(positions[:, 0].astype(f32), (0, npd))
    py = jnp.pad(positions[:, 1].astype(f32), (0, npd))
    pz = jnp.pad(positions[:, 2].astype(f32), (0, npd))
    pad_e = EPAD - N_EDGES
    src = jnp.concatenate([edge_index[0], jnp.full((pad_e,), N_NODES, jnp.int32)])
    dst = jnp.concatenate([edge_index[1], jnp.full((pad_e,), N_NODES, jnp.int32)])
    wpack = jnp.concatenate([
        W1.astype(f32).ravel(), b1.astype(f32), W2.astype(f32).ravel(),
        b2.astype(f32), jnp.zeros((15,), f32)])
    zeros1 = jnp.zeros((NPAD,), f32)
    log_pad = jnp.pad(local_or_ghost.astype(f32), (0, npd))
    batch_pad = jnp.pad(batch, (0, npd))

    partials = _edge_kernel(px, py, pz, src, dst, wpack, zeros1)
    final, tpart = _node_kernel(partials, log_pad, batch_pad)

    total_energy = tpart[:NUM_GRAPHS] + tpart[NUM_GRAPHS:]
    node_energy = final[:N_NODES]
    forces = jnp.stack([final[NPAD:NPAD + N_NODES],
                        final[2 * NPAD:2 * NPAD + N_NODES],
                        final[3 * NPAD:3 * NPAD + N_NODES]], axis=1)
    virials = jnp.zeros((1, 3, 3), dtype=positions.dtype)
    return total_energy, node_energy, forces, virials


# speedup vs baseline: 10.7956x; 1.1712x over previous
import functools

import jax
import jax.numpy as jnp
from jax import lax
from jax.experimental import pallas as pl
from jax.experimental.pallas import tpu as pltpu
from jax.experimental.pallas import tpu_sc as plsc

N_NODES = 100000
N_EDGES = 1600000
NUM_GRAPHS = 64
N_RBF = 8
HIDDEN = 16

NPAD = 100352
CHUNK = 1024
CHUNKS_PER_TILE = 49
NBLK = CHUNK // 128
EPT = CHUNK * CHUNKS_PER_TILE
EPAD = EPT * 32
CENTERS = [0.0 + 5.0 * k / (N_RBF - 1) for k in range(N_RBF)]

_mesh = plsc.VectorSubcoreMesh(core_axis_name="c", subcore_axis_name="s")
_params = pltpu.CompilerParams(needs_layout_passes=False,
                               use_tc_tiling_on_sc=False)


def _bcast_i32(x):
    return jnp.broadcast_to(jnp.int32(x), (16,))


def _rsqrt(u):
    bits = plsc.bitcast(u, jnp.int32)
    y = plsc.bitcast(jnp.int32(0x5F3759DF) - (bits >> 1), jnp.float32)
    for _ in range(3):
        y = y * (1.5 - 0.5 * u * y * y)
    return y


@functools.partial(
    pl.kernel,
    out_type=jax.ShapeDtypeStruct((8 * NPAD,), jnp.float32),
    mesh=_mesh,
    compiler_params=_params,
    scratch_types=[
        pltpu.VMEM((176,), jnp.float32),
        pltpu.VMEM((NBLK, 128), jnp.int32),
        pltpu.VMEM((NBLK, 128), jnp.int32),
        pltpu.VMEM((CHUNK,), jnp.float32),
        pltpu.VMEM((CHUNK,), jnp.float32),
        pltpu.VMEM((CHUNK,), jnp.float32),
        pltpu.VMEM((CHUNK,), jnp.float32),
        pltpu.VMEM((CHUNK,), jnp.float32),
        pltpu.VMEM((CHUNK,), jnp.float32),
        pltpu.VMEM((CHUNK,), jnp.float32),
        pltpu.VMEM((CHUNK,), jnp.float32),
        pltpu.VMEM((CHUNK,), jnp.float32),
        pltpu.VMEM((CHUNK,), jnp.float32),
        pltpu.VMEM((CHUNK,), jnp.float32),
        pltpu.VMEM((CHUNK,), jnp.float32),
        pltpu.VMEM((CHUNK,), jnp.float32),
        pltpu.VMEM_SHARED((NPAD,), jnp.float32),
        pltpu.VMEM_SHARED((NPAD,), jnp.float32),
        pltpu.VMEM_SHARED((NPAD,), jnp.float32),
        pltpu.VMEM_SHARED((NPAD,), jnp.float32),
        pltpu.SemaphoreType.DMA,
    ],
)
def _edge_kernel(px_h, py_h, pz_h, srcix_h, dstix_h, wpack_h, zeros_h, out_h,
                 wp, sidx, didx, sxb, syb, szb, dxb, dyb, dzb,
                 eb, gxb, gyb, gzb, nxb, nyb, nzb,
                 ae, agx, agy, agz, sem):
    cid = lax.axis_index("c")
    sid = lax.axis_index("s")
    wid = cid * 16 + sid

    zr = NPAD // 16
    zsl = pl.ds(sid * zr, zr)
    pltpu.sync_copy(zeros_h.at[zsl], ae.at[zsl])
    pltpu.sync_copy(zeros_h.at[zsl], agx.at[zsl])
    pltpu.sync_copy(zeros_h.at[zsl], agy.at[zsl])
    pltpu.sync_copy(zeros_h.at[zsl], agz.at[zsl])
    pltpu.sync_copy(wpack_h, wp)
    plsc.subcore_barrier()

    wvecs = [wp[pl.ds(i * 16, 16)] for i in range(11)]
    s_w1 = [[wvecs[k][jj] for jj in range(16)] for k in range(N_RBF)]
    s_b1 = [wvecs[8][jj] for jj in range(16)]
    s_w2 = [wvecs[9][jj] for jj in range(16)]
    s_b2 = wvecs[10][0]

    def chunk_body(c, carry):
        rb = wid * (CHUNKS_PER_TILE * NBLK) + c * NBLK
        cpi = [pltpu.async_copy(srcix_h.at[pl.ds(rb, NBLK)], sidx, sem),
               pltpu.async_copy(dstix_h.at[pl.ds(rb, NBLK)], didx, sem)]
        for cp in cpi:
            cp.wait()
        cps = []
        for j in range(NBLK):
            bsl = pl.ds(j * 128, 128)
            cps.append(pltpu.async_copy(px_h.at[sidx.at[j]], sxb.at[bsl], sem))
            cps.append(pltpu.async_copy(py_h.at[sidx.at[j]], syb.at[bsl], sem))
            cps.append(pltpu.async_copy(pz_h.at[sidx.at[j]], szb.at[bsl], sem))
            cps.append(pltpu.async_copy(px_h.at[didx.at[j]], dxb.at[bsl], sem))
            cps.append(pltpu.async_copy(py_h.at[didx.at[j]], dyb.at[bsl], sem))
            cps.append(pltpu.async_copy(pz_h.at[didx.at[j]], dzb.at[bsl], sem))
        for cp in cps:
            cp.wait()

        def grp(g, carry2):
            gsl = pl.ds(g * 16, 16)
            vx = dxb[gsl] - sxb[gsl]
            vy = dyb[gsl] - syb[gsl]
            vz = dzb[gsl] - szb[gsl]
            u = vx * vx + vy * vy + vz * vz + 1e-9
            inv_d = _rsqrt(u)
            d = u * inv_d

            rbf = []
            drbf = []
            for k in range(N_RBF):
                t = d - CENTERS[k]
                r = jnp.exp(-(t * t))
                rbf.append(r)
                drbf.append(-2.0 * t * r)

            e_vec = None
            dd_vec = None
            for jj in range(HIDDEN):
                zv = rbf[0] * s_w1[0][jj]
                gv = drbf[0] * s_w1[0][jj]
                for k in range(1, N_RBF):
                    zv = zv + rbf[k] * s_w1[k][jj]
                    gv = gv + drbf[k] * s_w1[k][jj]
                zv = zv + s_b1[jj]
                ex = jnp.exp(zv + zv)
                th = (ex - 1.0) / (ex + 1.0)
                w2s = s_w2[jj]
                et = th * w2s
                dt = (1.0 - th * th) * gv * w2s
                e_vec = et if e_vec is None else e_vec + et
                dd_vec = dt if dd_vec is None else dd_vec + dt
            e_vec = e_vec + s_b2

            coef = dd_vec * inv_d
            gx = coef * vx
            gy = coef * vy
            gz = coef * vz

            eb[gsl] = e_vec
            gxb[gsl] = gx
            gyb[gsl] = gy
            gzb[gsl] = gz
            nxb[gsl] = -gx
            nyb[gsl] = -gy
            nzb[gsl] = -gz
            return carry2

        lax.fori_loop(0, CHUNK // 16, grp, 0)

        css = []
        for j in range(NBLK):
            bsl = pl.ds(j * 128, 128)
            css.append(pltpu.async_copy(eb.at[bsl], ae.at[didx.at[j]], sem, add=True))
            css.append(pltpu.async_copy(gxb.at[bsl], agx.at[didx.at[j]], sem, add=True))
            css.append(pltpu.async_copy(gyb.at[bsl], agy.at[didx.at[j]], sem, add=True))
            css.append(pltpu.async_copy(gzb.at[bsl], agz.at[didx.at[j]], sem, add=True))
            css.append(pltpu.async_copy(nxb.at[bsl], agx.at[sidx.at[j]], sem, add=True))
            css.append(pltpu.async_copy(nyb.at[bsl], agy.at[sidx.at[j]], sem, add=True))
            css.append(pltpu.async_copy(nzb.at[bsl], agz.at[sidx.at[j]], sem, add=True))
        for cp in css:
            cp.wait()
        return carry

    lax.fori_loop(0, CHUNKS_PER_TILE, chunk_body, 0)

    plsc.subcore_barrier()
    obase = cid * 4 * NPAD
    pltpu.sync_copy(ae.at[zsl], out_h.at[pl.ds(obase + sid * zr, zr)])
    pltpu.sync_copy(agx.at[zsl], out_h.at[pl.ds(obase + NPAD + sid * zr, zr)])
    pltpu.sync_copy(agy.at[zsl], out_h.at[pl.ds(obase + 2 * NPAD + sid * zr, zr)])
    pltpu.sync_copy(agz.at[zsl], out_h.at[pl.ds(obase + 3 * NPAD + sid * zr, zr)])


_ROWS_B = NPAD // 32


@functools.partial(
    pl.kernel,
    out_type=(
        jax.ShapeDtypeStruct((4 * NPAD,), jnp.float32),
        jax.ShapeDtypeStruct((2 * NUM_GRAPHS,), jnp.float32),
    ),
    mesh=_mesh,
    compiler_params=_params,
    scratch_types=[
        pltpu.VMEM((_ROWS_B,), jnp.float32),
        pltpu.VMEM((_ROWS_B,), jnp.float32),
        pltpu.VMEM((_ROWS_B,), jnp.float32),
        pltpu.VMEM((_ROWS_B,), jnp.float32),
        pltpu.VMEM((_ROWS_B,), jnp.float32),
        pltpu.VMEM((_ROWS_B,), jnp.float32),
        pltpu.VMEM((_ROWS_B,), jnp.float32),
        pltpu.VMEM((_ROWS_B,), jnp.float32),
        pltpu.VMEM((_ROWS_B,), jnp.float32),
        pltpu.VMEM((_ROWS_B,), jnp.int32),
        pltpu.VMEM((_ROWS_B,), jnp.float32),
        pltpu.VMEM((_ROWS_B,), jnp.float32),
        pltpu.VMEM((_ROWS_B,), jnp.float32),
        pltpu.VMEM((_ROWS_B,), jnp.float32),
        pltpu.VMEM((NUM_GRAPHS * 16,), jnp.float32),
        pltpu.VMEM((16 * NUM_GRAPHS * 16,), jnp.float32),
        pltpu.VMEM((NUM_GRAPHS,), jnp.float32),
        pltpu.VMEM_SHARED((16 * NUM_GRAPHS * 16,), jnp.float32),
    ],
)
def _node_kernel(p_h, log_h, batch_h, out_h, tot_h,
                 pae, pax, pay, paz, pbe, pbx, pby, pbz, lg, bt,
                 oe, ofx, ofy, ofz, bkt, tbuf, t64, sbkt):
    cid = lax.axis_index("c")
    sid = lax.axis_index("s")
    wid = cid * 16 + sid
    nb = wid * _ROWS_B
    sl = pl.ds(nb, _ROWS_B)

    pltpu.sync_copy(p_h.at[pl.ds(0 * NPAD + nb, _ROWS_B)], pae)
    pltpu.sync_copy(p_h.at[pl.ds(1 * NPAD + nb, _ROWS_B)], pax)
    pltpu.sync_copy(p_h.at[pl.ds(2 * NPAD + nb, _ROWS_B)], pay)
    pltpu.sync_copy(p_h.at[pl.ds(3 * NPAD + nb, _ROWS_B)], paz)
    pltpu.sync_copy(p_h.at[pl.ds(4 * NPAD + nb, _ROWS_B)], pbe)
    pltpu.sync_copy(p_h.at[pl.ds(5 * NPAD + nb, _ROWS_B)], pbx)
    pltpu.sync_copy(p_h.at[pl.ds(6 * NPAD + nb, _ROWS_B)], pby)
    pltpu.sync_copy(p_h.at[pl.ds(7 * NPAD + nb, _ROWS_B)], pbz)
    pltpu.sync_copy(log_h.at[sl], lg)
    pltpu.sync_copy(batch_h.at[sl], bt)

    lane = lax.iota(jnp.int32, 16)
    zero_v = jnp.broadcast_to(jnp.float32(0.0), (16,))

    def zb(i, carry):
        bkt[pl.ds(i * 16, 16)] = zero_v
        return carry

    lax.fori_loop(0, NUM_GRAPHS, zb, 0)

    def grp(g, carry):
        gsl = pl.ds(g * 16, 16)
        e = pae[gsl] + pbe[gsl]
        fx = -(pax[gsl] + pbx[gsl])
        fy = -(pay[gsl] + pby[gsl])
        fz = -(paz[gsl] + pbz[gsl])
        nel = e * lg[gsl]
        bv = bt[gsl]
        plsc.addupdate_scatter(bkt, [bv * 16 + lane], nel)
        oe[gsl] = e
        ofx[gsl] = fx
        ofy[gsl] = fy
        ofz[gsl] = fz
        return carry

    lax.fori_loop(0, _ROWS_B // 16, grp, 0)

    pltpu.sync_copy(oe, out_h.at[pl.ds(0 * NPAD + nb, _ROWS_B)])
    pltpu.sync_copy(ofx, out_h.at[pl.ds(1 * NPAD + nb, _ROWS_B)])
    pltpu.sync_copy(ofy, out_h.at[pl.ds(2 * NPAD + nb, _ROWS_B)])
    pltpu.sync_copy(ofz, out_h.at[pl.ds(3 * NPAD + nb, _ROWS_B)])
    pltpu.sync_copy(bkt, sbkt.at[pl.ds(sid * NUM_GRAPHS * 16, NUM_GRAPHS * 16)])
    plsc.subcore_barrier()

    @pl.when(sid == 0)
    def _():
        pltpu.sync_copy(sbkt, tbuf)
        for jg in range(NUM_GRAPHS):
            s = tbuf[pl.ds(jg * 16, 16)]
            for t in range(1, 16):
                s = s + tbuf[pl.ds(t * NUM_GRAPHS * 16 + jg * 16, 16)]
            plsc.store_scatter(t64, [_bcast_i32(jg)],
                               jnp.broadcast_to(jnp.sum(s), (16,)))
        pltpu.sync_copy(t64, tot_h.at[pl.ds(cid * NUM_GRAPHS, NUM_GRAPHS)])


def kernel(positions, local_or_ghost, W1, b1, W2, b2, edge_index, batch, ptr):
    f32 = jnp.float32
    npd = NPAD - N_NODES
    px = jnp.pad(positions[:, 0].astype(f32), (0, npd))
    py = jnp.pad(positions[:, 1].astype(f32), (0, npd))
    pz = jnp.pad(positions[:, 2].astype(f32), (0, npd))
    pad_e = EPAD - N_EDGES
    src = jnp.concatenate(
        [edge_index[0], jnp.full((pad_e,), N_NODES, jnp.int32)]).reshape(-1, 128)
    dst = jnp.concatenate(
        [edge_index[1], jnp.full((pad_e,), N_NODES, jnp.int32)]).reshape(-1, 128)
    wpack = jnp.concatenate([
        W1.astype(f32).ravel(), b1.astype(f32), W2.astype(f32).ravel(),
        b2.astype(f32), jnp.zeros((15,), f32)])
    zeros1 = jnp.zeros((NPAD,), f32)
    log_pad = jnp.pad(local_or_ghost.astype(f32), (0, npd))
    batch_pad = jnp.pad(batch, (0, npd))

    partials = _edge_kernel(px, py, pz, src, dst, wpack, zeros1)
    final, tpart = _node_kernel(partials, log_pad, batch_pad)

    total_energy = tpart[:NUM_GRAPHS] + tpart[NUM_GRAPHS:]
    node_energy = final[:N_NODES]
    forces = jnp.stack([final[NPAD:NPAD + N_NODES],
                        final[2 * NPAD:2 * NPAD + N_NODES],
                        final[3 * NPAD:3 * NPAD + N_NODES]], axis=1)
    virials = jnp.zeros((1, 3, 3), dtype=positions.dtype)
    return total_energy, node_energy, forces, virials


# speedup vs baseline: 11.9958x; 1.1112x over previous
import functools

import jax
import jax.numpy as jnp
from jax import lax
from jax.experimental import pallas as pl
from jax.experimental.pallas import tpu as pltpu
from jax.experimental.pallas import tpu_sc as plsc

N_NODES = 100000
N_EDGES = 1600000
NUM_GRAPHS = 64
N_RBF = 8
HIDDEN = 16

NPAD = 100352
CHUNK = 1024
CHUNKS_PER_TILE = 49
NBLK = CHUNK // 128
EPT = CHUNK * CHUNKS_PER_TILE
EPAD = EPT * 32
CENTERS = [0.0 + 5.0 * k / (N_RBF - 1) for k in range(N_RBF)]

_mesh = plsc.VectorSubcoreMesh(core_axis_name="c", subcore_axis_name="s")
_params = pltpu.CompilerParams(needs_layout_passes=False,
                               use_tc_tiling_on_sc=False)


def _bcast_i32(x):
    return jnp.broadcast_to(jnp.int32(x), (16,))


def _rsqrt(u):
    bits = plsc.bitcast(u, jnp.int32)
    y = plsc.bitcast(jnp.int32(0x5F3759DF) - (bits >> 1), jnp.float32)
    for _ in range(3):
        y = y * (1.5 - 0.5 * u * y * y)
    return y


@functools.partial(
    pl.kernel,
    out_type=jax.ShapeDtypeStruct((8 * NPAD,), jnp.float32),
    mesh=_mesh,
    compiler_params=_params,
    scratch_types=[
        pltpu.VMEM((176,), jnp.float32),
        pltpu.VMEM((NBLK, 128), jnp.int32),
        pltpu.VMEM((NBLK, 128), jnp.int32),
        pltpu.VMEM((CHUNK,), jnp.float32),
        pltpu.VMEM((CHUNK,), jnp.float32),
        pltpu.VMEM((CHUNK,), jnp.float32),
        pltpu.VMEM((CHUNK,), jnp.float32),
        pltpu.VMEM((CHUNK,), jnp.float32),
        pltpu.VMEM((CHUNK,), jnp.float32),
        pltpu.VMEM((CHUNK,), jnp.float32),
        pltpu.VMEM((CHUNK,), jnp.float32),
        pltpu.VMEM((CHUNK,), jnp.float32),
        pltpu.VMEM((CHUNK,), jnp.float32),
        pltpu.VMEM((CHUNK,), jnp.float32),
        pltpu.VMEM((CHUNK,), jnp.float32),
        pltpu.VMEM((CHUNK,), jnp.float32),
        pltpu.VMEM_SHARED((NPAD,), jnp.float32),
        pltpu.VMEM_SHARED((NPAD,), jnp.float32),
        pltpu.VMEM_SHARED((NPAD,), jnp.float32),
        pltpu.VMEM_SHARED((NPAD,), jnp.float32),
        pltpu.VMEM_SHARED((NPAD,), jnp.float32),
        pltpu.VMEM_SHARED((NPAD,), jnp.float32),
        pltpu.VMEM_SHARED((NPAD,), jnp.float32),
        pltpu.SemaphoreType.DMA,
    ],
)
def _edge_kernel(px_h, py_h, pz_h, srcix_h, dstix_h, wpack_h, zeros_h, out_h,
                 wp, sidx, didx, sxb, syb, szb, dxb, dyb, dzb,
                 eb, gxb, gyb, gzb, nxb, nyb, nzb,
                 spx, spy, spz, ae, agx, agy, agz, sem):
    cid = lax.axis_index("c")
    sid = lax.axis_index("s")
    wid = cid * 16 + sid

    zr = NPAD // 16
    zsl = pl.ds(sid * zr, zr)
    pltpu.sync_copy(px_h.at[zsl], spx.at[zsl])
    pltpu.sync_copy(py_h.at[zsl], spy.at[zsl])
    pltpu.sync_copy(pz_h.at[zsl], spz.at[zsl])
    pltpu.sync_copy(zeros_h.at[zsl], ae.at[zsl])
    pltpu.sync_copy(zeros_h.at[zsl], agx.at[zsl])
    pltpu.sync_copy(zeros_h.at[zsl], agy.at[zsl])
    pltpu.sync_copy(zeros_h.at[zsl], agz.at[zsl])
    pltpu.sync_copy(wpack_h, wp)
    plsc.subcore_barrier()

    wvecs = [wp[pl.ds(i * 16, 16)] for i in range(11)]
    s_w1 = [[wvecs[k][jj] for jj in range(16)] for k in range(N_RBF)]
    s_b1 = [wvecs[8][jj] for jj in range(16)]
    s_w2 = [wvecs[9][jj] for jj in range(16)]
    s_b2 = wvecs[10][0]

    def chunk_body(c, carry):
        rb = wid * (CHUNKS_PER_TILE * NBLK) + c * NBLK
        cpi = [pltpu.async_copy(srcix_h.at[pl.ds(rb, NBLK)], sidx, sem),
               pltpu.async_copy(dstix_h.at[pl.ds(rb, NBLK)], didx, sem)]
        for cp in cpi:
            cp.wait()
        cps = []
        for j in range(NBLK):
            bsl = pl.ds(j * 128, 128)
            cps.append(pltpu.async_copy(spx.at[sidx.at[j]], sxb.at[bsl], sem))
            cps.append(pltpu.async_copy(spy.at[sidx.at[j]], syb.at[bsl], sem))
            cps.append(pltpu.async_copy(spz.at[sidx.at[j]], szb.at[bsl], sem))
            cps.append(pltpu.async_copy(spx.at[didx.at[j]], dxb.at[bsl], sem))
            cps.append(pltpu.async_copy(spy.at[didx.at[j]], dyb.at[bsl], sem))
            cps.append(pltpu.async_copy(spz.at[didx.at[j]], dzb.at[bsl], sem))
        for cp in cps:
            cp.wait()

        def grp(g, carry2):
            gsl = pl.ds(g * 16, 16)
            vx = dxb[gsl] - sxb[gsl]
            vy = dyb[gsl] - syb[gsl]
            vz = dzb[gsl] - szb[gsl]
            u = vx * vx + vy * vy + vz * vz + 1e-9
            inv_d = _rsqrt(u)
            d = u * inv_d

            rbf = []
            drbf = []
            for k in range(N_RBF):
                t = d - CENTERS[k]
                r = jnp.exp(-(t * t))
                rbf.append(r)
                drbf.append(-2.0 * t * r)

            e_vec = None
            dd_vec = None
            for jj in range(HIDDEN):
                zv = rbf[0] * s_w1[0][jj]
                gv = drbf[0] * s_w1[0][jj]
                for k in range(1, N_RBF):
                    zv = zv + rbf[k] * s_w1[k][jj]
                    gv = gv + drbf[k] * s_w1[k][jj]
                zv = zv + s_b1[jj]
                ex = jnp.exp(zv + zv)
                th = (ex - 1.0) / (ex + 1.0)
                w2s = s_w2[jj]
                et = th * w2s
                dt = (1.0 - th * th) * gv * w2s
                e_vec = et if e_vec is None else e_vec + et
                dd_vec = dt if dd_vec is None else dd_vec + dt
            e_vec = e_vec + s_b2

            coef = dd_vec * inv_d
            gx = coef * vx
            gy = coef * vy
            gz = coef * vz

            eb[gsl] = e_vec
            gxb[gsl] = gx
            gyb[gsl] = gy
            gzb[gsl] = gz
            nxb[gsl] = -gx
            nyb[gsl] = -gy
            nzb[gsl] = -gz
            return carry2

        lax.fori_loop(0, CHUNK // 16, grp, 0)

        css = []
        for j in range(NBLK):
            bsl = pl.ds(j * 128, 128)
            css.append(pltpu.async_copy(eb.at[bsl], ae.at[didx.at[j]], sem, add=True))
            css.append(pltpu.async_copy(gxb.at[bsl], agx.at[didx.at[j]], sem, add=True))
            css.append(pltpu.async_copy(gyb.at[bsl], agy.at[didx.at[j]], sem, add=True))
            css.append(pltpu.async_copy(gzb.at[bsl], agz.at[didx.at[j]], sem, add=True))
            css.append(pltpu.async_copy(nxb.at[bsl], agx.at[sidx.at[j]], sem, add=True))
            css.append(pltpu.async_copy(nyb.at[bsl], agy.at[sidx.at[j]], sem, add=True))
            css.append(pltpu.async_copy(nzb.at[bsl], agz.at[sidx.at[j]], sem, add=True))
        for cp in css:
            cp.wait()
        return carry

    lax.fori_loop(0, CHUNKS_PER_TILE, chunk_body, 0)

    plsc.subcore_barrier()
    obase = cid * 4 * NPAD
    pltpu.sync_copy(ae.at[zsl], out_h.at[pl.ds(obase + sid * zr, zr)])
    pltpu.sync_copy(agx.at[zsl], out_h.at[pl.ds(obase + NPAD + sid * zr, zr)])
    pltpu.sync_copy(agy.at[zsl], out_h.at[pl.ds(obase + 2 * NPAD + sid * zr, zr)])
    pltpu.sync_copy(agz.at[zsl], out_h.at[pl.ds(obase + 3 * NPAD + sid * zr, zr)])


_ROWS_B = NPAD // 32


@functools.partial(
    pl.kernel,
    out_type=(
        jax.ShapeDtypeStruct((4 * NPAD,), jnp.float32),
        jax.ShapeDtypeStruct((2 * NUM_GRAPHS,), jnp.float32),
    ),
    mesh=_mesh,
    compiler_params=_params,
    scratch_types=[
        pltpu.VMEM((_ROWS_B,), jnp.float32),
        pltpu.VMEM((_ROWS_B,), jnp.float32),
        pltpu.VMEM((_ROWS_B,), jnp.float32),
        pltpu.VMEM((_ROWS_B,), jnp.float32),
        pltpu.VMEM((_ROWS_B,), jnp.float32),
        pltpu.VMEM((_ROWS_B,), jnp.float32),
        pltpu.VMEM((_ROWS_B,), jnp.float32),
        pltpu.VMEM((_ROWS_B,), jnp.float32),
        pltpu.VMEM((_ROWS_B,), jnp.float32),
        pltpu.VMEM((_ROWS_B,), jnp.int32),
        pltpu.VMEM((_ROWS_B,), jnp.float32),
        pltpu.VMEM((_ROWS_B,), jnp.float32),
        pltpu.VMEM((_ROWS_B,), jnp.float32),
        pltpu.VMEM((_ROWS_B,), jnp.float32),
        pltpu.VMEM((NUM_GRAPHS * 16,), jnp.float32),
        pltpu.VMEM((16 * NUM_GRAPHS * 16,), jnp.float32),
        pltpu.VMEM((NUM_GRAPHS,), jnp.float32),
        pltpu.VMEM_SHARED((16 * NUM_GRAPHS * 16,), jnp.float32),
    ],
)
def _node_kernel(p_h, log_h, batch_h, out_h, tot_h,
                 pae, pax, pay, paz, pbe, pbx, pby, pbz, lg, bt,
                 oe, ofx, ofy, ofz, bkt, tbuf, t64, sbkt):
    cid = lax.axis_index("c")
    sid = lax.axis_index("s")
    wid = cid * 16 + sid
    nb = wid * _ROWS_B
    sl = pl.ds(nb, _ROWS_B)

    pltpu.sync_copy(p_h.at[pl.ds(0 * NPAD + nb, _ROWS_B)], pae)
    pltpu.sync_copy(p_h.at[pl.ds(1 * NPAD + nb, _ROWS_B)], pax)
    pltpu.sync_copy(p_h.at[pl.ds(2 * NPAD + nb, _ROWS_B)], pay)
    pltpu.sync_copy(p_h.at[pl.ds(3 * NPAD + nb, _ROWS_B)], paz)
    pltpu.sync_copy(p_h.at[pl.ds(4 * NPAD + nb, _ROWS_B)], pbe)
    pltpu.sync_copy(p_h.at[pl.ds(5 * NPAD + nb, _ROWS_B)], pbx)
    pltpu.sync_copy(p_h.at[pl.ds(6 * NPAD + nb, _ROWS_B)], pby)
    pltpu.sync_copy(p_h.at[pl.ds(7 * NPAD + nb, _ROWS_B)], pbz)
    pltpu.sync_copy(log_h.at[sl], lg)
    pltpu.sync_copy(batch_h.at[sl], bt)

    lane = lax.iota(jnp.int32, 16)
    zero_v = jnp.broadcast_to(jnp.float32(0.0), (16,))

    def zb(i, carry):
        bkt[pl.ds(i * 16, 16)] = zero_v
        return carry

    lax.fori_loop(0, NUM_GRAPHS, zb, 0)

    def grp(g, carry):
        gsl = pl.ds(g * 16, 16)
        e = pae[gsl] + pbe[gsl]
        fx = -(pax[gsl] + pbx[gsl])
        fy = -(pay[gsl] + pby[gsl])
        fz = -(paz[gsl] + pbz[gsl])
        nel = e * lg[gsl]
        bv = bt[gsl]
        plsc.addupdate_scatter(bkt, [bv * 16 + lane], nel)
        oe[gsl] = e
        ofx[gsl] = fx
        ofy[gsl] = fy
        ofz[gsl] = fz
        return carry

    lax.fori_loop(0, _ROWS_B // 16, grp, 0)

    pltpu.sync_copy(oe, out_h.at[pl.ds(0 * NPAD + nb, _ROWS_B)])
    pltpu.sync_copy(ofx, out_h.at[pl.ds(1 * NPAD + nb, _ROWS_B)])
    pltpu.sync_copy(ofy, out_h.at[pl.ds(2 * NPAD + nb, _ROWS_B)])
    pltpu.sync_copy(ofz, out_h.at[pl.ds(3 * NPAD + nb, _ROWS_B)])
    pltpu.sync_copy(bkt, sbkt.at[pl.ds(sid * NUM_GRAPHS * 16, NUM_GRAPHS * 16)])
    plsc.subcore_barrier()

    @pl.when(sid == 0)
    def _():
        pltpu.sync_copy(sbkt, tbuf)
        for jg in range(NUM_GRAPHS):
            s = tbuf[pl.ds(jg * 16, 16)]
            for t in range(1, 16):
                s = s + tbuf[pl.ds(t * NUM_GRAPHS * 16 + jg * 16, 16)]
            plsc.store_scatter(t64, [_bcast_i32(jg)],
                               jnp.broadcast_to(jnp.sum(s), (16,)))
        pltpu.sync_copy(t64, tot_h.at[pl.ds(cid * NUM_GRAPHS, NUM_GRAPHS)])


def kernel(positions, local_or_ghost, W1, b1, W2, b2, edge_index, batch, ptr):
    f32 = jnp.float32
    npd = NPAD - N_NODES
    px = jnp.pad(positions[:, 0].astype(f32), (0, npd))
    py = jnp.pad(positions[:, 1].astype(f32), (0, npd))
    pz = jnp.pad(positions[:, 2].astype(f32), (0, npd))
    pad_e = EPAD - N_EDGES
    src = jnp.concatenate(
        [edge_index[0], jnp.full((pad_e,), N_NODES, jnp.int32)]).reshape(-1, 128)
    dst = jnp.concatenate(
        [edge_index[1], jnp.full((pad_e,), N_NODES, jnp.int32)]).reshape(-1, 128)
    wpack = jnp.concatenate([
        W1.astype(f32).ravel(), b1.astype(f32), W2.astype(f32).ravel(),
        b2.astype(f32), jnp.zeros((15,), f32)])
    zeros1 = jnp.zeros((NPAD,), f32)
    log_pad = jnp.pad(local_or_ghost.astype(f32), (0, npd))
    batch_pad = jnp.pad(batch, (0, npd))

    partials = _edge_kernel(px, py, pz, src, dst, wpack, zeros1)
    final, tpart = _node_kernel(partials, log_pad, batch_pad)

    total_energy = tpart[:NUM_GRAPHS] + tpart[NUM_GRAPHS:]
    node_energy = final[:N_NODES]
    forces = jnp.stack([final[NPAD:NPAD + N_NODES],
                        final[2 * NPAD:2 * NPAD + N_NODES],
                        final[3 * NPAD:3 * NPAD + N_NODES]], axis=1)
    virials = jnp.zeros((1, 3, 3), dtype=positions.dtype)
    return total_energy, node_energy, forces, virials


# speedup vs baseline: 26.2820x; 2.1909x over previous
import functools

import jax
import jax.numpy as jnp
from jax import lax
from jax.experimental import pallas as pl
from jax.experimental.pallas import tpu as pltpu
from jax.experimental.pallas import tpu_sc as plsc

N_NODES = 100000
N_EDGES = 1600000
NUM_GRAPHS = 64
N_RBF = 8
HIDDEN = 16

NPAD = 100352
CHUNK = 1024
CHUNKS_PER_TILE = 49
NBLK = CHUNK // 128
EPT = CHUNK * CHUNKS_PER_TILE
EPAD = EPT * 32
CENTERS = [0.0 + 5.0 * k / (N_RBF - 1) for k in range(N_RBF)]

_mesh = plsc.VectorSubcoreMesh(core_axis_name="c", subcore_axis_name="s")
_params = pltpu.CompilerParams(needs_layout_passes=False,
                               use_tc_tiling_on_sc=False)


def _bcast_i32(x):
    return jnp.broadcast_to(jnp.int32(x), (16,))


def _rsqrt(u):
    bits = plsc.bitcast(u, jnp.int32)
    y = plsc.bitcast(jnp.int32(0x5F3759DF) - (bits >> 1), jnp.float32)
    for _ in range(3):
        y = y * (1.5 - 0.5 * u * y * y)
    return y


@functools.partial(
    pl.kernel,
    out_type=jax.ShapeDtypeStruct((8 * NPAD,), jnp.float32),
    mesh=_mesh,
    compiler_params=_params,
    scratch_types=[
        pltpu.VMEM((176,), jnp.float32),
        pltpu.VMEM((NBLK, 128), jnp.int32),
        pltpu.VMEM((NBLK, 128), jnp.int32),
        pltpu.VMEM((CHUNK,), jnp.float32),
        pltpu.VMEM((CHUNK,), jnp.float32),
        pltpu.VMEM((CHUNK,), jnp.float32),
        pltpu.VMEM((CHUNK,), jnp.float32),
        pltpu.VMEM((CHUNK,), jnp.float32),
        pltpu.VMEM((CHUNK,), jnp.float32),
        pltpu.VMEM((CHUNK,), jnp.float32),
        pltpu.VMEM((CHUNK,), jnp.float32),
        pltpu.VMEM((CHUNK,), jnp.float32),
        pltpu.VMEM((CHUNK,), jnp.float32),
        pltpu.VMEM((CHUNK,), jnp.float32),
        pltpu.VMEM((CHUNK,), jnp.float32),
        pltpu.VMEM((CHUNK,), jnp.float32),
        pltpu.VMEM_SHARED((NPAD,), jnp.float32),
        pltpu.VMEM_SHARED((NPAD,), jnp.float32),
        pltpu.VMEM_SHARED((NPAD,), jnp.float32),
        pltpu.VMEM_SHARED((NPAD,), jnp.float32),
        pltpu.VMEM_SHARED((NPAD,), jnp.float32),
        pltpu.VMEM_SHARED((NPAD,), jnp.float32),
        pltpu.VMEM_SHARED((NPAD,), jnp.float32),
        pltpu.SemaphoreType.DMA,
    ],
)
def _edge_kernel(px_h, py_h, pz_h, srcix_h, dstix_h, wpack_h, zeros_h, out_h,
                 wp, sidx, didx, sxb, syb, szb, dxb, dyb, dzb,
                 eb, gxb, gyb, gzb, nxb, nyb, nzb,
                 spx, spy, spz, ae, agx, agy, agz, sem):
    cid = lax.axis_index("c")
    sid = lax.axis_index("s")
    wid = cid * 16 + sid

    zr = NPAD // 16
    zsl = pl.ds(sid * zr, zr)
    pltpu.sync_copy(px_h.at[zsl], spx.at[zsl])
    pltpu.sync_copy(py_h.at[zsl], spy.at[zsl])
    pltpu.sync_copy(pz_h.at[zsl], spz.at[zsl])
    pltpu.sync_copy(zeros_h.at[zsl], ae.at[zsl])
    pltpu.sync_copy(zeros_h.at[zsl], agx.at[zsl])
    pltpu.sync_copy(zeros_h.at[zsl], agy.at[zsl])
    pltpu.sync_copy(zeros_h.at[zsl], agz.at[zsl])
    pltpu.sync_copy(wpack_h, wp)
    plsc.subcore_barrier()

    wvecs = [wp[pl.ds(i * 16, 16)] for i in range(11)]
    s_w1 = [[wvecs[k][jj] for jj in range(16)] for k in range(N_RBF)]
    s_b1 = [wvecs[8][jj] for jj in range(16)]
    s_w2 = [wvecs[9][jj] for jj in range(16)]
    s_b2 = wvecs[10][0]

    def chunk_body(c, carry):
        rb = wid * (CHUNKS_PER_TILE * NBLK) + c * NBLK
        cpi = [pltpu.async_copy(srcix_h.at[pl.ds(rb, NBLK)], sidx, sem),
               pltpu.async_copy(dstix_h.at[pl.ds(rb, NBLK)], didx, sem)]
        for cp in cpi:
            cp.wait()
        cps = []
        for j in range(NBLK):
            bsl = pl.ds(j * 128, 128)
            cps.append(pltpu.async_copy(spx.at[sidx.at[j]], sxb.at[bsl], sem))
            cps.append(pltpu.async_copy(spy.at[sidx.at[j]], syb.at[bsl], sem))
            cps.append(pltpu.async_copy(spz.at[sidx.at[j]], szb.at[bsl], sem))
            cps.append(pltpu.async_copy(spx.at[didx.at[j]], dxb.at[bsl], sem))
            cps.append(pltpu.async_copy(spy.at[didx.at[j]], dyb.at[bsl], sem))
            cps.append(pltpu.async_copy(spz.at[didx.at[j]], dzb.at[bsl], sem))
        for cp in cps:
            cp.wait()

        @plsc.parallel_loop(0, CHUNK // 16, unroll=2)
        def grp(g):
            gsl = pl.ds(g * 16, 16)
            vx = dxb[gsl] - sxb[gsl]
            vy = dyb[gsl] - syb[gsl]
            vz = dzb[gsl] - szb[gsl]
            u = vx * vx + vy * vy + vz * vz + 1e-9
            inv_d = _rsqrt(u)
            d = u * inv_d

            rbf = []
            drbf = []
            for k in range(N_RBF):
                t = d - CENTERS[k]
                r = jnp.exp(-(t * t))
                rbf.append(r)
                drbf.append(-2.0 * t * r)

            e_vec = None
            dd_vec = None
            for jj in range(HIDDEN):
                zv = rbf[0] * s_w1[0][jj]
                gv = drbf[0] * s_w1[0][jj]
                for k in range(1, N_RBF):
                    zv = zv + rbf[k] * s_w1[k][jj]
                    gv = gv + drbf[k] * s_w1[k][jj]
                zv = zv + s_b1[jj]
                ex = jnp.exp(zv + zv)
                th = (ex - 1.0) / (ex + 1.0)
                w2s = s_w2[jj]
                et = th * w2s
                dt = (1.0 - th * th) * gv * w2s
                e_vec = et if e_vec is None else e_vec + et
                dd_vec = dt if dd_vec is None else dd_vec + dt
            e_vec = e_vec + s_b2

            coef = dd_vec * inv_d
            gx = coef * vx
            gy = coef * vy
            gz = coef * vz

            eb[gsl] = e_vec
            gxb[gsl] = gx
            gyb[gsl] = gy
            gzb[gsl] = gz
            nxb[gsl] = -gx
            nyb[gsl] = -gy
            nzb[gsl] = -gz

        css = []
        for j in range(NBLK):
            bsl = pl.ds(j * 128, 128)
            css.append(pltpu.async_copy(eb.at[bsl], ae.at[didx.at[j]], sem, add=True))
            css.append(pltpu.async_copy(gxb.at[bsl], agx.at[didx.at[j]], sem, add=True))
            css.append(pltpu.async_copy(gyb.at[bsl], agy.at[didx.at[j]], sem, add=True))
            css.append(pltpu.async_copy(gzb.at[bsl], agz.at[didx.at[j]], sem, add=True))
            css.append(pltpu.async_copy(nxb.at[bsl], agx.at[sidx.at[j]], sem, add=True))
            css.append(pltpu.async_copy(nyb.at[bsl], agy.at[sidx.at[j]], sem, add=True))
            css.append(pltpu.async_copy(nzb.at[bsl], agz.at[sidx.at[j]], sem, add=True))
        for cp in css:
            cp.wait()
        return carry

    lax.fori_loop(0, CHUNKS_PER_TILE, chunk_body, 0)

    plsc.subcore_barrier()
    obase = cid * 4 * NPAD
    pltpu.sync_copy(ae.at[zsl], out_h.at[pl.ds(obase + sid * zr, zr)])
    pltpu.sync_copy(agx.at[zsl], out_h.at[pl.ds(obase + NPAD + sid * zr, zr)])
    pltpu.sync_copy(agy.at[zsl], out_h.at[pl.ds(obase + 2 * NPAD + sid * zr, zr)])
    pltpu.sync_copy(agz.at[zsl], out_h.at[pl.ds(obase + 3 * NPAD + sid * zr, zr)])


_ROWS_B = NPAD // 32


@functools.partial(
    pl.kernel,
    out_type=(
        jax.ShapeDtypeStruct((4 * NPAD,), jnp.float32),
        jax.ShapeDtypeStruct((2 * NUM_GRAPHS,), jnp.float32),
    ),
    mesh=_mesh,
    compiler_params=_params,
    scratch_types=[
        pltpu.VMEM((_ROWS_B,), jnp.float32),
        pltpu.VMEM((_ROWS_B,), jnp.float32),
        pltpu.VMEM((_ROWS_B,), jnp.float32),
        pltpu.VMEM((_ROWS_B,), jnp.float32),
        pltpu.VMEM((_ROWS_B,), jnp.float32),
        pltpu.VMEM((_ROWS_B,), jnp.float32),
        pltpu.VMEM((_ROWS_B,), jnp.float32),
        pltpu.VMEM((_ROWS_B,), jnp.float32),
        pltpu.VMEM((_ROWS_B,), jnp.float32),
        pltpu.VMEM((_ROWS_B,), jnp.int32),
        pltpu.VMEM((_ROWS_B,), jnp.float32),
        pltpu.VMEM((_ROWS_B,), jnp.float32),
        pltpu.VMEM((_ROWS_B,), jnp.float32),
        pltpu.VMEM((_ROWS_B,), jnp.float32),
        pltpu.VMEM((NUM_GRAPHS * 16,), jnp.float32),
        pltpu.VMEM((16 * NUM_GRAPHS * 16,), jnp.float32),
        pltpu.VMEM((NUM_GRAPHS,), jnp.float32),
        pltpu.VMEM_SHARED((16 * NUM_GRAPHS * 16,), jnp.float32),
    ],
)
def _node_kernel(p_h, log_h, batch_h, out_h, tot_h,
                 pae, pax, pay, paz, pbe, pbx, pby, pbz, lg, bt,
                 oe, ofx, ofy, ofz, bkt, tbuf, t64, sbkt):
    cid = lax.axis_index("c")
    sid = lax.axis_index("s")
    wid = cid * 16 + sid
    nb = wid * _ROWS_B
    sl = pl.ds(nb, _ROWS_B)

    pltpu.sync_copy(p_h.at[pl.ds(0 * NPAD + nb, _ROWS_B)], pae)
    pltpu.sync_copy(p_h.at[pl.ds(1 * NPAD + nb, _ROWS_B)], pax)
    pltpu.sync_copy(p_h.at[pl.ds(2 * NPAD + nb, _ROWS_B)], pay)
    pltpu.sync_copy(p_h.at[pl.ds(3 * NPAD + nb, _ROWS_B)], paz)
    pltpu.sync_copy(p_h.at[pl.ds(4 * NPAD + nb, _ROWS_B)], pbe)
    pltpu.sync_copy(p_h.at[pl.ds(5 * NPAD + nb, _ROWS_B)], pbx)
    pltpu.sync_copy(p_h.at[pl.ds(6 * NPAD + nb, _ROWS_B)], pby)
    pltpu.sync_copy(p_h.at[pl.ds(7 * NPAD + nb, _ROWS_B)], pbz)
    pltpu.sync_copy(log_h.at[sl], lg)
    pltpu.sync_copy(batch_h.at[sl], bt)

    lane = lax.iota(jnp.int32, 16)
    zero_v = jnp.broadcast_to(jnp.float32(0.0), (16,))

    def zb(i, carry):
        bkt[pl.ds(i * 16, 16)] = zero_v
        return carry

    lax.fori_loop(0, NUM_GRAPHS, zb, 0)

    def grp(g, carry):
        gsl = pl.ds(g * 16, 16)
        e = pae[gsl] + pbe[gsl]
        fx = -(pax[gsl] + pbx[gsl])
        fy = -(pay[gsl] + pby[gsl])
        fz = -(paz[gsl] + pbz[gsl])
        nel = e * lg[gsl]
        bv = bt[gsl]
        plsc.addupdate_scatter(bkt, [bv * 16 + lane], nel)
        oe[gsl] = e
        ofx[gsl] = fx
        ofy[gsl] = fy
        ofz[gsl] = fz
        return carry

    lax.fori_loop(0, _ROWS_B // 16, grp, 0)

    pltpu.sync_copy(oe, out_h.at[pl.ds(0 * NPAD + nb, _ROWS_B)])
    pltpu.sync_copy(ofx, out_h.at[pl.ds(1 * NPAD + nb, _ROWS_B)])
    pltpu.sync_copy(ofy, out_h.at[pl.ds(2 * NPAD + nb, _ROWS_B)])
    pltpu.sync_copy(ofz, out_h.at[pl.ds(3 * NPAD + nb, _ROWS_B)])
    pltpu.sync_copy(bkt, sbkt.at[pl.ds(sid * NUM_GRAPHS * 16, NUM_GRAPHS * 16)])
    plsc.subcore_barrier()

    @pl.when(sid == 0)
    def _():
        pltpu.sync_copy(sbkt, tbuf)
        for jg in range(NUM_GRAPHS):
            s = tbuf[pl.ds(jg * 16, 16)]
            for t in range(1, 16):
                s = s + tbuf[pl.ds(t * NUM_GRAPHS * 16 + jg * 16, 16)]
            plsc.store_scatter(t64, [_bcast_i32(jg)],
                               jnp.broadcast_to(jnp.sum(s), (16,)))
        pltpu.sync_copy(t64, tot_h.at[pl.ds(cid * NUM_GRAPHS, NUM_GRAPHS)])


def kernel(positions, local_or_ghost, W1, b1, W2, b2, edge_index, batch, ptr):
    f32 = jnp.float32
    npd = NPAD - N_NODES
    px = jnp.pad(positions[:, 0].astype(f32), (0, npd))
    py = jnp.pad(positions[:, 1].astype(f32), (0, npd))
    pz = jnp.pad(positions[:, 2].astype(f32), (0, npd))
    pad_e = EPAD - N_EDGES
    src = jnp.concatenate(
        [edge_index[0], jnp.full((pad_e,), N_NODES, jnp.int32)]).reshape(-1, 128)
    dst = jnp.concatenate(
        [edge_index[1], jnp.full((pad_e,), N_NODES, jnp.int32)]).reshape(-1, 128)
    wpack = jnp.concatenate([
        W1.astype(f32).ravel(), b1.astype(f32), W2.astype(f32).ravel(),
        b2.astype(f32), jnp.zeros((15,), f32)])
    zeros1 = jnp.zeros((NPAD,), f32)
    log_pad = jnp.pad(local_or_ghost.astype(f32), (0, npd))
    batch_pad = jnp.pad(batch, (0, npd))

    partials = _edge_kernel(px, py, pz, src, dst, wpack, zeros1)
    final, tpart = _node_kernel(partials, log_pad, batch_pad)

    total_energy = tpart[:NUM_GRAPHS] + tpart[NUM_GRAPHS:]
    node_energy = final[:N_NODES]
    forces = jnp.stack([final[NPAD:NPAD + N_NODES],
                        final[2 * NPAD:2 * NPAD + N_NODES],
                        final[3 * NPAD:3 * NPAD + N_NODES]], axis=1)
    virials = jnp.zeros((1, 3, 3), dtype=positions.dtype)
    return total_energy, node_energy, forces, virials


# speedup vs baseline: 27.6090x; 1.0505x over previous
import functools

import jax
import jax.numpy as jnp
from jax import lax
from jax.experimental import pallas as pl
from jax.experimental.pallas import tpu as pltpu
from jax.experimental.pallas import tpu_sc as plsc

N_NODES = 100000
N_EDGES = 1600000
NUM_GRAPHS = 64
N_RBF = 8
HIDDEN = 16

NPAD = 100352
CHUNK = 1024
CHUNKS_PER_TILE = 49
NBLK = CHUNK // 128
EPT = CHUNK * CHUNKS_PER_TILE
EPAD = EPT * 32
CENTERS = [0.0 + 5.0 * k / (N_RBF - 1) for k in range(N_RBF)]

_mesh = plsc.VectorSubcoreMesh(core_axis_name="c", subcore_axis_name="s")
_params = pltpu.CompilerParams(needs_layout_passes=False,
                               use_tc_tiling_on_sc=False)


def _bcast_i32(x):
    return jnp.broadcast_to(jnp.int32(x), (16,))


def _rsqrt(u):
    bits = plsc.bitcast(u, jnp.int32)
    y = plsc.bitcast(jnp.int32(0x5F3759DF) - (bits >> 1), jnp.float32)
    for _ in range(3):
        y = y * (1.5 - 0.5 * u * y * y)
    return y


@functools.partial(
    pl.kernel,
    out_type=jax.ShapeDtypeStruct((8 * NPAD,), jnp.float32),
    mesh=_mesh,
    compiler_params=_params,
    scratch_types=[
        pltpu.VMEM((176,), jnp.float32),
        pltpu.VMEM((NBLK, 128), jnp.int32),
        pltpu.VMEM((NBLK, 128), jnp.int32),
        pltpu.VMEM((CHUNK,), jnp.float32),
        pltpu.VMEM((CHUNK,), jnp.float32),
        pltpu.VMEM((CHUNK,), jnp.float32),
        pltpu.VMEM((CHUNK,), jnp.float32),
        pltpu.VMEM((CHUNK,), jnp.float32),
        pltpu.VMEM((CHUNK,), jnp.float32),
        pltpu.VMEM((CHUNK,), jnp.float32),
        pltpu.VMEM((CHUNK,), jnp.float32),
        pltpu.VMEM((CHUNK,), jnp.float32),
        pltpu.VMEM((CHUNK,), jnp.float32),
        pltpu.VMEM((CHUNK,), jnp.float32),
        pltpu.VMEM((CHUNK,), jnp.float32),
        pltpu.VMEM((CHUNK,), jnp.float32),
        pltpu.VMEM_SHARED((NPAD,), jnp.float32),
        pltpu.VMEM_SHARED((NPAD,), jnp.float32),
        pltpu.VMEM_SHARED((NPAD,), jnp.float32),
        pltpu.VMEM_SHARED((NPAD,), jnp.float32),
        pltpu.VMEM_SHARED((NPAD,), jnp.float32),
        pltpu.VMEM_SHARED((NPAD,), jnp.float32),
        pltpu.VMEM_SHARED((NPAD,), jnp.float32),
        pltpu.SemaphoreType.DMA,
    ],
)
def _edge_kernel(px_h, py_h, pz_h, srcix_h, dstix_h, wpack_h, zeros_h, out_h,
                 wp, sidx, didx, sxb, syb, szb, dxb, dyb, dzb,
                 eb, gxb, gyb, gzb, nxb, nyb, nzb,
                 spx, spy, spz, ae, agx, agy, agz, sem):
    cid = lax.axis_index("c")
    sid = lax.axis_index("s")
    wid = cid * 16 + sid

    zr = NPAD // 16
    zsl = pl.ds(sid * zr, zr)
    pltpu.sync_copy(px_h.at[zsl], spx.at[zsl])
    pltpu.sync_copy(py_h.at[zsl], spy.at[zsl])
    pltpu.sync_copy(pz_h.at[zsl], spz.at[zsl])
    pltpu.sync_copy(zeros_h.at[zsl], ae.at[zsl])
    pltpu.sync_copy(zeros_h.at[zsl], agx.at[zsl])
    pltpu.sync_copy(zeros_h.at[zsl], agy.at[zsl])
    pltpu.sync_copy(zeros_h.at[zsl], agz.at[zsl])
    pltpu.sync_copy(wpack_h, wp)
    plsc.subcore_barrier()

    wvecs = [wp[pl.ds(i * 16, 16)] for i in range(11)]
    s_w1 = [[wvecs[k][jj] for jj in range(16)] for k in range(N_RBF)]
    s_b1 = [wvecs[8][jj] for jj in range(16)]
    s_w2 = [wvecs[9][jj] for jj in range(16)]
    s_b2 = wvecs[10][0]

    def chunk_body(c, carry):
        rb = wid * (CHUNKS_PER_TILE * NBLK) + c * NBLK
        cpi = [pltpu.async_copy(srcix_h.at[pl.ds(rb, NBLK)], sidx, sem),
               pltpu.async_copy(dstix_h.at[pl.ds(rb, NBLK)], didx, sem)]
        for cp in cpi:
            cp.wait()
        cps = []
        for j in range(NBLK):
            bsl = pl.ds(j * 128, 128)
            cps.append(pltpu.async_copy(spx.at[sidx.at[j]], sxb.at[bsl], sem))
            cps.append(pltpu.async_copy(spy.at[sidx.at[j]], syb.at[bsl], sem))
            cps.append(pltpu.async_copy(spz.at[sidx.at[j]], szb.at[bsl], sem))
            cps.append(pltpu.async_copy(spx.at[didx.at[j]], dxb.at[bsl], sem))
            cps.append(pltpu.async_copy(spy.at[didx.at[j]], dyb.at[bsl], sem))
            cps.append(pltpu.async_copy(spz.at[didx.at[j]], dzb.at[bsl], sem))
        for cp in cps:
            cp.wait()

        @plsc.parallel_loop(0, CHUNK // 16, unroll=4)
        def grp(g):
            gsl = pl.ds(g * 16, 16)
            vx = dxb[gsl] - sxb[gsl]
            vy = dyb[gsl] - syb[gsl]
            vz = dzb[gsl] - szb[gsl]
            u = vx * vx + vy * vy + vz * vz + 1e-9
            inv_d = _rsqrt(u)
            d = u * inv_d

            rbf = []
            drbf = []
            for k in range(N_RBF):
                t = d - CENTERS[k]
                r = jnp.exp(-(t * t))
                rbf.append(r)
                drbf.append(-2.0 * t * r)

            e_vec = None
            dd_vec = None
            for jj in range(HIDDEN):
                zv = rbf[0] * s_w1[0][jj]
                gv = drbf[0] * s_w1[0][jj]
                for k in range(1, N_RBF):
                    zv = zv + rbf[k] * s_w1[k][jj]
                    gv = gv + drbf[k] * s_w1[k][jj]
                zv = zv + s_b1[jj]
                ex = jnp.exp(zv + zv)
                th = (ex - 1.0) / (ex + 1.0)
                w2s = s_w2[jj]
                et = th * w2s
                dt = (1.0 - th * th) * gv * w2s
                e_vec = et if e_vec is None else e_vec + et
                dd_vec = dt if dd_vec is None else dd_vec + dt
            e_vec = e_vec + s_b2

            coef = dd_vec * inv_d
            gx = coef * vx
            gy = coef * vy
            gz = coef * vz

            eb[gsl] = e_vec
            gxb[gsl] = gx
            gyb[gsl] = gy
            gzb[gsl] = gz
            nxb[gsl] = -gx
            nyb[gsl] = -gy
            nzb[gsl] = -gz

        css = []
        for j in range(NBLK):
            bsl = pl.ds(j * 128, 128)
            css.append(pltpu.async_copy(eb.at[bsl], ae.at[didx.at[j]], sem, add=True))
            css.append(pltpu.async_copy(gxb.at[bsl], agx.at[didx.at[j]], sem, add=True))
            css.append(pltpu.async_copy(gyb.at[bsl], agy.at[didx.at[j]], sem, add=True))
            css.append(pltpu.async_copy(gzb.at[bsl], agz.at[didx.at[j]], sem, add=True))
            css.append(pltpu.async_copy(nxb.at[bsl], agx.at[sidx.at[j]], sem, add=True))
            css.append(pltpu.async_copy(nyb.at[bsl], agy.at[sidx.at[j]], sem, add=True))
            css.append(pltpu.async_copy(nzb.at[bsl], agz.at[sidx.at[j]], sem, add=True))
        for cp in css:
            cp.wait()
        return carry

    lax.fori_loop(0, CHUNKS_PER_TILE, chunk_body, 0)

    plsc.subcore_barrier()
    obase = cid * 4 * NPAD
    pltpu.sync_copy(ae.at[zsl], out_h.at[pl.ds(obase + sid * zr, zr)])
    pltpu.sync_copy(agx.at[zsl], out_h.at[pl.ds(obase + NPAD + sid * zr, zr)])
    pltpu.sync_copy(agy.at[zsl], out_h.at[pl.ds(obase + 2 * NPAD + sid * zr, zr)])
    pltpu.sync_copy(agz.at[zsl], out_h.at[pl.ds(obase + 3 * NPAD + sid * zr, zr)])


_ROWS_B = NPAD // 32


@functools.partial(
    pl.kernel,
    out_type=(
        jax.ShapeDtypeStruct((4 * NPAD,), jnp.float32),
        jax.ShapeDtypeStruct((2 * NUM_GRAPHS,), jnp.float32),
    ),
    mesh=_mesh,
    compiler_params=_params,
    scratch_types=[
        pltpu.VMEM((_ROWS_B,), jnp.float32),
        pltpu.VMEM((_ROWS_B,), jnp.float32),
        pltpu.VMEM((_ROWS_B,), jnp.float32),
        pltpu.VMEM((_ROWS_B,), jnp.float32),
        pltpu.VMEM((_ROWS_B,), jnp.float32),
        pltpu.VMEM((_ROWS_B,), jnp.float32),
        pltpu.VMEM((_ROWS_B,), jnp.float32),
        pltpu.VMEM((_ROWS_B,), jnp.float32),
        pltpu.VMEM((_ROWS_B,), jnp.float32),
        pltpu.VMEM((_ROWS_B,), jnp.int32),
        pltpu.VMEM((_ROWS_B,), jnp.float32),
        pltpu.VMEM((_ROWS_B,), jnp.float32),
        pltpu.VMEM((_ROWS_B,), jnp.float32),
        pltpu.VMEM((_ROWS_B,), jnp.float32),
        pltpu.VMEM((NUM_GRAPHS * 16,), jnp.float32),
        pltpu.VMEM((16 * NUM_GRAPHS * 16,), jnp.float32),
        pltpu.VMEM((NUM_GRAPHS,), jnp.float32),
        pltpu.VMEM_SHARED((16 * NUM_GRAPHS * 16,), jnp.float32),
    ],
)
def _node_kernel(p_h, log_h, batch_h, out_h, tot_h,
                 pae, pax, pay, paz, pbe, pbx, pby, pbz, lg, bt,
                 oe, ofx, ofy, ofz, bkt, tbuf, t64, sbkt):
    cid = lax.axis_index("c")
    sid = lax.axis_index("s")
    wid = cid * 16 + sid
    nb = wid * _ROWS_B
    sl = pl.ds(nb, _ROWS_B)

    pltpu.sync_copy(p_h.at[pl.ds(0 * NPAD + nb, _ROWS_B)], pae)
    pltpu.sync_copy(p_h.at[pl.ds(1 * NPAD + nb, _ROWS_B)], pax)
    pltpu.sync_copy(p_h.at[pl.ds(2 * NPAD + nb, _ROWS_B)], pay)
    pltpu.sync_copy(p_h.at[pl.ds(3 * NPAD + nb, _ROWS_B)], paz)
    pltpu.sync_copy(p_h.at[pl.ds(4 * NPAD + nb, _ROWS_B)], pbe)
    pltpu.sync_copy(p_h.at[pl.ds(5 * NPAD + nb, _ROWS_B)], pbx)
    pltpu.sync_copy(p_h.at[pl.ds(6 * NPAD + nb, _ROWS_B)], pby)
    pltpu.sync_copy(p_h.at[pl.ds(7 * NPAD + nb, _ROWS_B)], pbz)
    pltpu.sync_copy(log_h.at[sl], lg)
    pltpu.sync_copy(batch_h.at[sl], bt)

    lane = lax.iota(jnp.int32, 16)
    zero_v = jnp.broadcast_to(jnp.float32(0.0), (16,))

    def zb(i, carry):
        bkt[pl.ds(i * 16, 16)] = zero_v
        return carry

    lax.fori_loop(0, NUM_GRAPHS, zb, 0)

    def grp(g, carry):
        gsl = pl.ds(g * 16, 16)
        e = pae[gsl] + pbe[gsl]
        fx = -(pax[gsl] + pbx[gsl])
        fy = -(pay[gsl] + pby[gsl])
        fz = -(paz[gsl] + pbz[gsl])
        nel = e * lg[gsl]
        bv = bt[gsl]
        plsc.addupdate_scatter(bkt, [bv * 16 + lane], nel)
        oe[gsl] = e
        ofx[gsl] = fx
        ofy[gsl] = fy
        ofz[gsl] = fz
        return carry

    lax.fori_loop(0, _ROWS_B // 16, grp, 0)

    pltpu.sync_copy(oe, out_h.at[pl.ds(0 * NPAD + nb, _ROWS_B)])
    pltpu.sync_copy(ofx, out_h.at[pl.ds(1 * NPAD + nb, _ROWS_B)])
    pltpu.sync_copy(ofy, out_h.at[pl.ds(2 * NPAD + nb, _ROWS_B)])
    pltpu.sync_copy(ofz, out_h.at[pl.ds(3 * NPAD + nb, _ROWS_B)])
    pltpu.sync_copy(bkt, sbkt.at[pl.ds(sid * NUM_GRAPHS * 16, NUM_GRAPHS * 16)])
    plsc.subcore_barrier()

    @pl.when(sid == 0)
    def _():
        pltpu.sync_copy(sbkt, tbuf)
        for jg in range(NUM_GRAPHS):
            s = tbuf[pl.ds(jg * 16, 16)]
            for t in range(1, 16):
                s = s + tbuf[pl.ds(t * NUM_GRAPHS * 16 + jg * 16, 16)]
            plsc.store_scatter(t64, [_bcast_i32(jg)],
                               jnp.broadcast_to(jnp.sum(s), (16,)))
        pltpu.sync_copy(t64, tot_h.at[pl.ds(cid * NUM_GRAPHS, NUM_GRAPHS)])


def kernel(positions, local_or_ghost, W1, b1, W2, b2, edge_index, batch, ptr):
    f32 = jnp.float32
    npd = NPAD - N_NODES
    px = jnp.pad(positions[:, 0].astype(f32), (0, npd))
    py = jnp.pad(positions[:, 1].astype(f32), (0, npd))
    pz = jnp.pad(positions[:, 2].astype(f32), (0, npd))
    pad_e = EPAD - N_EDGES
    src = jnp.concatenate(
        [edge_index[0], jnp.full((pad_e,), N_NODES, jnp.int32)]).reshape(-1, 128)
    dst = jnp.concatenate(
        [edge_index[1], jnp.full((pad_e,), N_NODES, jnp.int32)]).reshape(-1, 128)
    wpack = jnp.concatenate([
        W1.astype(f32).ravel(), b1.astype(f32), W2.astype(f32).ravel(),
        b2.astype(f32), jnp.zeros((15,), f32)])
    zeros1 = jnp.zeros((NPAD,), f32)
    log_pad = jnp.pad(local_or_ghost.astype(f32), (0, npd))
    batch_pad = jnp.pad(batch, (0, npd))

    partials = _edge_kernel(px, py, pz, src, dst, wpack, zeros1)
    final, tpart = _node_kernel(partials, log_pad, batch_pad)

    total_energy = tpart[:NUM_GRAPHS] + tpart[NUM_GRAPHS:]
    node_energy = final[:N_NODES]
    forces = jnp.stack([final[NPAD:NPAD + N_NODES],
                        final[2 * NPAD:2 * NPAD + N_NODES],
                        final[3 * NPAD:3 * NPAD + N_NODES]], axis=1)
    virials = jnp.zeros((1, 3, 3), dtype=positions.dtype)
    return total_energy, node_energy, forces, virials
